# baseline (speedup 1.0000x reference)
import functools

import jax
import jax.numpy as jnp
from jax import lax
from jax.experimental import pallas as pl
from jax.experimental.pallas import tpu as pltpu
from jax.experimental.pallas import tpu_sc as plsc

N_NODES = 10000
N_EDGES = 320000
D_IN = 128
D_HID = 16

NC = 2
NS = 16
NW = NC * NS
EPW = N_EDGES // NW
K = 125
NCHUNK = EPW // K
RING = 10
RPS = N_NODES // NS
DEG_PAD = 10240
DEG_PS = DEG_PAD // NS

_mesh = plsc.VectorSubcoreMesh(core_axis_name="c", subcore_axis_name="s")
_sc_params = pltpu.CompilerParams(use_tc_tiling_on_sc=False)
_sc_params_nl = pltpu.CompilerParams(use_tc_tiling_on_sc=False,
                                     needs_layout_passes=False)


NCH2 = NC * NCHUNK
XPS = DEG_PAD // NW


@functools.partial(
    pl.kernel,
    out_type=jax.ShapeDtypeStruct((DEG_PAD, D_HID), jnp.float32),
    mesh=_mesh,
    scratch_types=[
        pltpu.VMEM((NCH2, K), jnp.int32),
        pltpu.VMEM((128,), jnp.float32),
        pltpu.VMEM((DEG_PS,), jnp.float32),
        pltpu.VMEM((XPS,), jnp.float32),
        pltpu.VMEM((XPS, D_HID), jnp.float32),
        pltpu.VMEM_SHARED((DEG_PAD,), jnp.float32),
        pltpu.SemaphoreType.DMA,
    ],
    compiler_params=_sc_params_nl,
)
def _deg_kernel(edges_hbm, out_hbm, idx_v, ones_v, zero_v, dinv_v, exp_v,
                acc_sh, ssem):
    cid = lax.axis_index("c")
    sid = lax.axis_index("s")

    @pl.loop(0, 128, step=16)
    def _(i):
        ones_v.at[pl.ds(i, 16)][...] = jnp.ones((16,), jnp.float32)

    @pl.loop(0, DEG_PS, step=16)
    def _(i):
        zero_v.at[pl.ds(i, 16)][...] = jnp.zeros((16,), jnp.float32)

    pltpu.sync_copy(zero_v, acc_sh.at[pl.ds(sid * DEG_PS, DEG_PS)])
    pltpu.sync_copy(edges_hbm.at[1, 2 * sid], idx_v.at[pl.ds(0, NCHUNK)])
    pltpu.sync_copy(edges_hbm.at[1, 2 * sid + 1],
                    idx_v.at[pl.ds(NCHUNK, NCHUNK)])
    plsc.subcore_barrier()

    ones_slc = ones_v.at[pl.ds(0, K)]

    @pl.loop(0, NCH2)
    def _(j):
        pltpu.async_copy(ones_slc, acc_sh.at[idx_v.at[j]], ssem, add=True)

    @pl.loop(0, NCH2)
    def _(j):
        pltpu.make_async_copy(ones_slc, acc_sh.at[idx_v.at[j]], ssem).wait()

    plsc.subcore_barrier()

    base = (cid * NS + sid) * XPS
    pltpu.sync_copy(acc_sh.at[pl.ds(base, XPS)], dinv_v)

    @pl.loop(0, XPS, step=16)
    def _(i):
        d = dinv_v.at[pl.ds(i, 16)][...] + 1.0
        bits = plsc.bitcast(d, jnp.int32)
        y = plsc.bitcast(
            jnp.int32(0x5F3759DF) - lax.shift_right_logical(bits, 1),
            jnp.float32)
        hd = 0.5 * d
        y = y * (1.5 - hd * y * y)
        y = y * (1.5 - hd * y * y)
        y = y * (1.5 - hd * y * y)
        dinv_v.at[pl.ds(i, 16)][...] = y

    @pl.loop(0, XPS)
    def _(n):
        idx16 = jnp.full((16,), n, jnp.int32)
        exp_v.at[n][...] = plsc.load_gather(dinv_v, [idx16])

    pltpu.sync_copy(exp_v, out_hbm.at[pl.ds(base, XPS)])


@functools.partial(
    pl.kernel,
    out_type=jax.ShapeDtypeStruct((NC, NS, RPS, D_HID), jnp.float32),
    mesh=_mesh,
    scratch_types=[
        pltpu.VMEM((NCHUNK, K), jnp.int32),
        pltpu.VMEM((NCHUNK, K), jnp.int32),
    ] + [pltpu.VMEM((K, D_HID), jnp.float32) for _ in range(RING)] + [
        pltpu.VMEM((RPS, D_HID), jnp.float32),
        pltpu.VMEM_SHARED((N_NODES, D_HID), jnp.float32),
        pltpu.VMEM_SHARED((N_NODES, D_HID), jnp.float32),
        pltpu.SemaphoreType.DMA((RING,)),
        pltpu.SemaphoreType.DMA((RING,)),
    ],
    compiler_params=_sc_params,
)
def _agg_kernel(table_hbm, edges_hbm, out_hbm,
                src_v, dst_v, r0, r1, r2, r3, r4, r5, r6, r7, r8, r9,
                zrows_v, acc_sh, table_sh, gsem, ssem):
    rows = (r0, r1, r2, r3, r4, r5, r6, r7, r8, r9)
    cid = lax.axis_index("c")
    sid = lax.axis_index("s")
    wid = sid * NC + cid

    @pl.loop(0, RPS)
    def _(r):
        zrows_v.at[r][...] = jnp.zeros((16,), jnp.float32)

    pltpu.sync_copy(zrows_v, acc_sh.at[pl.ds(sid * RPS, RPS)])
    pltpu.sync_copy(table_hbm.at[pl.ds(sid * RPS, RPS)],
                    table_sh.at[pl.ds(sid * RPS, RPS)])
    pltpu.sync_copy(edges_hbm.at[0, wid], src_v)
    pltpu.sync_copy(edges_hbm.at[1, wid], dst_v)
    plsc.subcore_barrier()

    def gather_start(c, b):
        pltpu.async_copy(table_sh.at[src_v.at[c]], rows[b], gsem.at[b])

    def gather_wait(c, b):
        pltpu.make_async_copy(table_sh.at[src_v.at[c]], rows[b],
                              gsem.at[b]).wait()

    def scat_start(c, b):
        pltpu.async_copy(rows[b], acc_sh.at[dst_v.at[c]], ssem.at[b],
                         add=True)

    def scat_wait(c, b):
        pltpu.make_async_copy(rows[b], acc_sh.at[dst_v.at[c]],
                              ssem.at[b]).wait()

    for b in range(RING):
        gather_start(b, b)

    @pl.loop(0, NCHUNK - RING, step=RING)
    def _(j):
        for b in range(RING):
            gather_wait(j + b, b)
            scat_start(j + b, b)
        for b in range(RING):
            scat_wait(j + b, b)
            gather_start(j + RING + b, b)

    for b in range(RING):
        gather_wait(NCHUNK - RING + b, b)
        scat_start(NCHUNK - RING + b, b)
    for b in range(RING):
        scat_wait(NCHUNK - RING + b, b)

    plsc.subcore_barrier()
    pltpu.sync_copy(acc_sh.at[pl.ds(sid * RPS, RPS)], out_hbm.at[cid, sid])


PK = 128 // D_HID
PROWS = N_NODES // PK


def _mm_body(x_ref, w_ref, o_ref):
    o_ref[...] = lax.dot_general(
        x_ref[...], w_ref[...], (((1,), (0,)), ((), ())),
        precision=lax.Precision.HIGHEST, preferred_element_type=jnp.float32)


def _scale1_body(xw_ref, dinvp_ref, yw_ref):
    yw_ref[...] = xw_ref[...] * dinvp_ref[0:PROWS, :]


def _mid_body(dinvp_ref, agg_ref, yw1_ref, b1_ref, w2_ref, yw2_ref):
    dinvp = dinvp_ref[0:PROWS, :]
    t = agg_ref[0] + agg_ref[1] + yw1_ref[...]
    h = jnp.maximum(t * dinvp + b1_ref[...], 0.0)
    hw = lax.dot_general(
        h, w2_ref[...], (((1,), (0,)), ((), ())),
        precision=lax.Precision.HIGHEST, preferred_element_type=jnp.float32)
    yw2_ref[...] = hw * dinvp


def _final_body(dinvp_ref, agg_ref, yw2_ref, b2_ref, out_ref):
    t = agg_ref[0] + agg_ref[1] + yw2_ref[...]
    out_ref[...] = t * dinvp_ref[0:PROWS, :] + b2_ref[...]


def kernel(x, edge_index, W1, b1, W2, b2):
    f32 = jnp.float32
    edges = edge_index.astype(jnp.int32).reshape(2, NW, NCHUNK, K)

    eye8 = jnp.eye(PK, dtype=f32)
    w1bd = jnp.kron(eye8, W1)
    w2bd = jnp.kron(eye8, W2)
    b1p = jnp.tile(b1, PK).reshape(1, 128)
    b2p = jnp.tile(b2, PK).reshape(1, 128)

    dinvp = _deg_kernel(edges).reshape(DEG_PAD // PK, 128)

    xw1p = pl.pallas_call(
        _mm_body,
        out_shape=jax.ShapeDtypeStruct((PROWS, 128), f32),
    )(x.reshape(PROWS, PK * D_IN), w1bd)

    yw1p = pl.pallas_call(
        _scale1_body,
        out_shape=jax.ShapeDtypeStruct((PROWS, 128), f32),
    )(xw1p, dinvp)

    agg1 = _agg_kernel(yw1p.reshape(N_NODES, D_HID), edges)
    agg1p = agg1.reshape(NC, PROWS, 128)

    yw2p = pl.pallas_call(
        _mid_body,
        out_shape=jax.ShapeDtypeStruct((PROWS, 128), f32),
    )(dinvp, agg1p, yw1p, b1p, w2bd)

    agg2 = _agg_kernel(yw2p.reshape(N_NODES, D_HID), edges)
    agg2p = agg2.reshape(NC, PROWS, 128)

    outp = pl.pallas_call(
        _final_body,
        out_shape=jax.ShapeDtypeStruct((PROWS, 128), f32),
    )(dinvp, agg2p, yw2p, b2p)
    return outp.reshape(N_NODES, D_HID)


# speedup vs baseline: 1.0578x; 1.0578x over previous
import functools

import jax
import jax.numpy as jnp
from jax import lax
from jax.experimental import pallas as pl
from jax.experimental.pallas import tpu as pltpu
from jax.experimental.pallas import tpu_sc as plsc

N_NODES = 10000
N_EDGES = 320000
D_IN = 128
D_HID = 16

NC = 2
NS = 16
NW = NC * NS
EPW = N_EDGES // NW
K = 125
NCHUNK = EPW // K
RING = 10
RPS = N_NODES // NS
DEG_PAD = 10240
DEG_PS = DEG_PAD // NS

_mesh = plsc.VectorSubcoreMesh(core_axis_name="c", subcore_axis_name="s")
_sc_params = pltpu.CompilerParams(use_tc_tiling_on_sc=False)
_sc_params_nl = pltpu.CompilerParams(use_tc_tiling_on_sc=False,
                                     needs_layout_passes=False)


NCH2 = NC * NCHUNK
XPS = DEG_PAD // NW


@functools.partial(
    pl.kernel,
    out_type=jax.ShapeDtypeStruct((DEG_PAD, D_HID), jnp.float32),
    mesh=_mesh,
    scratch_types=[
        pltpu.VMEM((NCH2, K), jnp.int32),
        pltpu.VMEM((128,), jnp.float32),
        pltpu.VMEM((DEG_PS,), jnp.float32),
        pltpu.VMEM((XPS,), jnp.float32),
        pltpu.VMEM((XPS, D_HID), jnp.float32),
        pltpu.VMEM_SHARED((DEG_PAD,), jnp.float32),
        pltpu.SemaphoreType.DMA,
    ],
    compiler_params=_sc_params_nl,
)
def _deg_kernel(edges_hbm, out_hbm, idx_v, ones_v, zero_v, dinv_v, exp_v,
                acc_sh, ssem):
    cid = lax.axis_index("c")
    sid = lax.axis_index("s")

    @pl.loop(0, 128, step=16)
    def _(i):
        ones_v.at[pl.ds(i, 16)][...] = jnp.ones((16,), jnp.float32)

    @pl.loop(0, DEG_PS, step=16)
    def _(i):
        zero_v.at[pl.ds(i, 16)][...] = jnp.zeros((16,), jnp.float32)

    pltpu.sync_copy(zero_v, acc_sh.at[pl.ds(sid * DEG_PS, DEG_PS)])
    pltpu.sync_copy(edges_hbm.at[1, 2 * sid], idx_v.at[pl.ds(0, NCHUNK)])
    pltpu.sync_copy(edges_hbm.at[1, 2 * sid + 1],
                    idx_v.at[pl.ds(NCHUNK, NCHUNK)])
    plsc.subcore_barrier()

    ones_slc = ones_v.at[pl.ds(0, K)]

    @pl.loop(0, NCH2)
    def _(j):
        pltpu.async_copy(ones_slc, acc_sh.at[idx_v.at[j]], ssem, add=True)

    @pl.loop(0, NCH2)
    def _(j):
        pltpu.make_async_copy(ones_slc, acc_sh.at[idx_v.at[j]], ssem).wait()

    plsc.subcore_barrier()

    base = (cid * NS + sid) * XPS
    pltpu.sync_copy(acc_sh.at[pl.ds(base, XPS)], dinv_v)

    @pl.loop(0, XPS, step=16)
    def _(i):
        d = dinv_v.at[pl.ds(i, 16)][...] + 1.0
        bits = plsc.bitcast(d, jnp.int32)
        y = plsc.bitcast(
            jnp.int32(0x5F3759DF) - lax.shift_right_logical(bits, 1),
            jnp.float32)
        hd = 0.5 * d
        y = y * (1.5 - hd * y * y)
        y = y * (1.5 - hd * y * y)
        y = y * (1.5 - hd * y * y)
        dinv_v.at[pl.ds(i, 16)][...] = y

    @pl.loop(0, XPS)
    def _(n):
        idx16 = jnp.full((16,), n, jnp.int32)
        exp_v.at[n][...] = plsc.load_gather(dinv_v, [idx16])

    pltpu.sync_copy(exp_v, out_hbm.at[pl.ds(base, XPS)])


@functools.partial(
    pl.kernel,
    out_type=jax.ShapeDtypeStruct((NC, NS, RPS, D_HID), jnp.float32),
    mesh=_mesh,
    scratch_types=[
        pltpu.VMEM((NCHUNK, K), jnp.int32),
        pltpu.VMEM((NCHUNK, K), jnp.int32),
    ] + [pltpu.VMEM((K, D_HID), jnp.float32) for _ in range(RING)] + [
        pltpu.VMEM((RPS, D_HID), jnp.float32),
        pltpu.VMEM_SHARED((N_NODES, D_HID), jnp.float32),
        pltpu.SemaphoreType.DMA((RING,)),
        pltpu.SemaphoreType.DMA((RING,)),
    ],
    compiler_params=_sc_params,
)
def _agg_kernel(table_hbm, edges_hbm, out_hbm,
                src_v, dst_v, r0, r1, r2, r3, r4, r5, r6, r7, r8, r9,
                zrows_v, acc_sh, gsem, ssem):
    rows = (r0, r1, r2, r3, r4, r5, r6, r7, r8, r9)
    cid = lax.axis_index("c")
    sid = lax.axis_index("s")
    wid = sid * NC + cid

    @pl.loop(0, RPS)
    def _(r):
        zrows_v.at[r][...] = jnp.zeros((16,), jnp.float32)

    pltpu.sync_copy(zrows_v, acc_sh.at[pl.ds(sid * RPS, RPS)])
    pltpu.sync_copy(edges_hbm.at[0, wid], src_v)
    pltpu.sync_copy(edges_hbm.at[1, wid], dst_v)
    plsc.subcore_barrier()

    def gather_start(c, b):
        pltpu.async_copy(table_hbm.at[src_v.at[c]], rows[b], gsem.at[b])

    def gather_wait(c, b):
        pltpu.make_async_copy(table_hbm.at[src_v.at[c]], rows[b],
                              gsem.at[b]).wait()

    def scat_start(c, b):
        pltpu.async_copy(rows[b], acc_sh.at[dst_v.at[c]], ssem.at[b],
                         add=True)

    def scat_wait(c, b):
        pltpu.make_async_copy(rows[b], acc_sh.at[dst_v.at[c]],
                              ssem.at[b]).wait()

    for b in range(RING):
        gather_start(b, b)

    @pl.loop(0, NCHUNK - RING, step=RING)
    def _(j):
        for b in range(RING):
            gather_wait(j + b, b)
            scat_start(j + b, b)
        for b in range(RING):
            scat_wait(j + b, b)
            gather_start(j + RING + b, b)

    for b in range(RING):
        gather_wait(NCHUNK - RING + b, b)
        scat_start(NCHUNK - RING + b, b)
    for b in range(RING):
        scat_wait(NCHUNK - RING + b, b)

    plsc.subcore_barrier()
    pltpu.sync_copy(acc_sh.at[pl.ds(sid * RPS, RPS)], out_hbm.at[cid, sid])


PK = 128 // D_HID
PROWS = N_NODES // PK


def _mm_body(x_ref, w_ref, o_ref):
    o_ref[...] = lax.dot_general(
        x_ref[...], w_ref[...], (((1,), (0,)), ((), ())),
        preferred_element_type=jnp.float32)


def _scale1_body(xw_ref, dinvp_ref, yw_ref):
    yw_ref[...] = xw_ref[...] * dinvp_ref[0:PROWS, :]


def _mid_body(dinvp_ref, agg_ref, yw1_ref, b1_ref, w2_ref, yw2_ref):
    dinvp = dinvp_ref[0:PROWS, :]
    t = agg_ref[0] + agg_ref[1] + yw1_ref[...]
    h = jnp.maximum(t * dinvp + b1_ref[...], 0.0)
    hw = lax.dot_general(
        h, w2_ref[...], (((1,), (0,)), ((), ())),
        precision=lax.Precision.HIGHEST, preferred_element_type=jnp.float32)
    yw2_ref[...] = hw * dinvp


def _final_body(dinvp_ref, agg_ref, yw2_ref, b2_ref, out_ref):
    t = agg_ref[0] + agg_ref[1] + yw2_ref[...]
    out_ref[...] = t * dinvp_ref[0:PROWS, :] + b2_ref[...]


def kernel(x, edge_index, W1, b1, W2, b2):
    f32 = jnp.float32
    edges = edge_index.astype(jnp.int32).reshape(2, NW, NCHUNK, K)

    eye8 = jnp.eye(PK, dtype=f32)
    w1bd = jnp.kron(eye8, W1)
    w2bd = jnp.kron(eye8, W2)
    b1p = jnp.tile(b1, PK).reshape(1, 128)
    b2p = jnp.tile(b2, PK).reshape(1, 128)

    dinvp = _deg_kernel(edges).reshape(DEG_PAD // PK, 128)

    xw1p = pl.pallas_call(
        _mm_body,
        out_shape=jax.ShapeDtypeStruct((PROWS, 128), f32),
    )(x.reshape(PROWS, PK * D_IN).astype(jnp.bfloat16),
      w1bd.astype(jnp.bfloat16))

    yw1p = pl.pallas_call(
        _scale1_body,
        out_shape=jax.ShapeDtypeStruct((PROWS, 128), f32),
    )(xw1p, dinvp)

    agg1 = _agg_kernel(yw1p.reshape(N_NODES, D_HID), edges)
    agg1p = agg1.reshape(NC, PROWS, 128)

    yw2p = pl.pallas_call(
        _mid_body,
        out_shape=jax.ShapeDtypeStruct((PROWS, 128), f32),
    )(dinvp, agg1p, yw1p, b1p, w2bd)

    agg2 = _agg_kernel(yw2p.reshape(N_NODES, D_HID), edges)
    agg2p = agg2.reshape(NC, PROWS, 128)

    outp = pl.pallas_call(
        _final_body,
        out_shape=jax.ShapeDtypeStruct((PROWS, 128), f32),
    )(dinvp, agg2p, yw2p, b2p)
    return outp.reshape(N_NODES, D_HID)
